# baseline (speedup 1.0000x reference)
import functools

import jax
import jax.numpy as jnp
from jax import lax
from jax.experimental import pallas as pl
from jax.experimental.pallas import tpu as pltpu
from jax.experimental.pallas import tpu_sc as plsc

_B = 16
_N = 2048
_H = 0.0005
_ROWS = 256

_NB_SC = 16
_NW = 32
_RPW = _NB_SC * _N // _NW


def _tc_body(xr_ref, xc_ref, out_ref):
    b = pl.program_id(0)
    r = pl.program_id(1)

    xr = xr_ref[0]
    xc = xc_ref[0]
    sq_r = jnp.sum(xr * xr, axis=1, keepdims=True)
    sq_c = jnp.sum(xc * xc, axis=0, keepdims=True)
    g = jax.lax.dot_general(
        xr.astype(jnp.bfloat16), xc.astype(jnp.bfloat16), (((1,), (0,)), ((), ())),
        preferred_element_type=jnp.float32,
    )
    d2 = jnp.maximum(sq_r + sq_c - 2.0 * g, 0.0)

    f = jnp.maximum(_H - d2 * d2, 0.0)
    iszero = d2 == 0.0
    n0 = jnp.sum(iszero.astype(jnp.float32), axis=1, keepdims=True)
    fp = jnp.where(iszero, 0.0, f)
    m1 = jnp.max(fp, axis=1, keepdims=True)
    f1 = jnp.where(fp == m1, 0.0, fp)
    m2 = jnp.max(f1, axis=1, keepdims=True)
    f2 = jnp.where(f1 == m2, 0.0, f1)
    m3 = jnp.max(f2, axis=1, keepdims=True)
    f3 = jnp.where(f2 == m3, 0.0, f2)
    m4 = jnp.max(f3, axis=1, keepdims=True)
    f4 = jnp.where(f3 == m4, 0.0, f3)
    m5 = jnp.max(f4, axis=1, keepdims=True)

    nz5 = jnp.minimum(n0, 5.0)
    npos = 5.0 - nz5
    s_pos = (jnp.where(npos >= 1, m1, 0.0) + jnp.where(npos >= 2, m2, 0.0)
             + jnp.where(npos >= 3, m3, 0.0) + jnp.where(npos >= 4, m4, 0.0)
             + jnp.where(npos >= 5, m5, 0.0))
    maxf = jnp.where(n0 > 0, _H, m1)
    contrib = jnp.sum(nz5 * _H + s_pos - maxf)

    @pl.when(jnp.logical_and(b == 0, r == 0))
    def _():
        out_ref[0, 0] = 0.0

    out_ref[0, 0] += contrib


def _tc_total(xt, pcd, nb):
    return pl.pallas_call(
        _tc_body,
        grid=(nb, _N // _ROWS),
        in_specs=[
            pl.BlockSpec((1, _ROWS, 3), lambda b, r: (b, r, 0)),
            pl.BlockSpec((1, 3, _N), lambda b, r: (b, 0, 0)),
        ],
        out_specs=pl.BlockSpec(memory_space=pltpu.SMEM),
        out_shape=jax.ShapeDtypeStruct((1, 1), jnp.float32),
    )(xt, pcd)[0, 0]


def _sc_body(xb_hbm, sq_hbm, out_hbm, x0_v, x1_v, x2_v, sq_v, frow_v, res_v, sem):
    wid = lax.axis_index("s") * 2 + lax.axis_index("c")
    base = wid * _RPW
    b = base // _N
    row0 = base % _N

    xoff = b * 3 * _N
    pltpu.sync_copy(xb_hbm.at[pl.ds(xoff, _N)], x0_v)
    pltpu.sync_copy(xb_hbm.at[pl.ds(xoff + _N, _N)], x1_v)
    pltpu.sync_copy(xb_hbm.at[pl.ds(xoff + 2 * _N, _N)], x2_v)
    pltpu.sync_copy(sq_hbm.at[pl.ds(b * _N, _N)], sq_v)

    def round_chunk(c, acc):
        sl = pl.ds(c * 16, 16)
        for ref in (x0_v, x1_v, x2_v):
            u = plsc.bitcast(ref[sl], jnp.uint32)
            u = (u + jnp.uint32(0x7FFF) + ((u >> jnp.uint32(16)) & jnp.uint32(1))) & jnp.uint32(0xFFFF0000)
            ref[sl] = plsc.bitcast(u, jnp.float32)
        return acc

    lax.fori_loop(0, _N // 16, round_chunk, 0)

    def row_step(i, total):
        n = row0 + i
        n16 = (n // 16) * 16
        onehot = jnp.where(lax.iota(jnp.int32, 16) == n - n16, 1.0, 0.0)
        slq = pl.ds(n16, 16)
        zero16 = jnp.zeros((16,), jnp.float32)

        def bcast(ref):
            return zero16 + lax.reduce_sum(ref[slq] * onehot, (0,))

        xn0 = bcast(x0_v)
        xn1 = bcast(x1_v)
        xn2 = bcast(x2_v)
        sqn = bcast(sq_v)

        def chunk_step(c, carry):
            s_vec, c_vec, z_vec, m_vec = carry
            sl = pl.ds(c * 16, 16)
            g = xn0 * x0_v[sl] + xn1 * x1_v[sl] + xn2 * x2_v[sl]
            d2 = jnp.maximum((sqn + sq_v[sl]) - 2.0 * g, 0.0)
            isz = d2 == 0.0
            f = jnp.maximum(_H - d2 * d2, 0.0)
            fp = jnp.where(isz, 0.0, f)
            frow_v[sl] = fp
            s_vec = s_vec + fp
            c_vec = c_vec + jnp.where(fp > 0.0, 1.0, 0.0)
            z_vec = z_vec + jnp.where(isz, 1.0, 0.0)
            m_vec = jnp.maximum(m_vec, fp)
            return s_vec, c_vec, z_vec, m_vec

        zero = jnp.zeros((16,), jnp.float32)
        s_vec, c_vec, z_vec, m_vec = lax.fori_loop(
            0, _N // 16, chunk_step, (zero, zero, zero, zero))
        s = lax.reduce_sum(s_vec, (0,))
        cnt = lax.reduce_sum(c_vec, (0,))
        n0 = lax.reduce_sum(z_vec, (0,))
        m1 = lax.reduce_max(m_vec, (0,))

        nz5 = jnp.minimum(n0, 5.0)
        npos = 5.0 - nz5

        def fast(_):
            return s

        def slow(_):
            def next_max(prev):
                def pass_step(c, acc):
                    sl = pl.ds(c * 16, 16)
                    v = frow_v[sl]
                    return jnp.maximum(acc, jnp.where(v < prev, v, 0.0))
                mv = lax.fori_loop(0, _N // 16, pass_step, zero)
                return lax.reduce_max(mv, (0,))
            m2 = next_max(m1)
            m3 = next_max(m2)
            m4 = next_max(m3)
            m5 = next_max(m4)
            return (jnp.where(npos >= 1, m1, 0.0) + jnp.where(npos >= 2, m2, 0.0)
                    + jnp.where(npos >= 3, m3, 0.0) + jnp.where(npos >= 4, m4, 0.0)
                    + jnp.where(npos >= 5, m5, 0.0))

        s_pos = lax.cond(cnt > npos, slow, fast, 0)
        maxf = jnp.where(n0 > 0.0, _H, m1)
        return total + (nz5 * _H + s_pos - maxf)

    total = lax.fori_loop(0, _RPW, row_step, jnp.float32(0.0))
    res_v[...] = jnp.where(lax.iota(jnp.int32, 16) == 0, total, 0.0)
    pltpu.sync_copy(res_v, out_hbm.at[pl.ds(wid * 16, 16)])


@functools.partial(
    pl.kernel,
    mesh=plsc.VectorSubcoreMesh(core_axis_name="c", subcore_axis_name="s"),
    out_type=jax.ShapeDtypeStruct((_NW * 16,), jnp.float32),
    compiler_params=pltpu.CompilerParams(needs_layout_passes=False),
    scratch_types=[
        pltpu.VMEM((_N,), jnp.float32),
        pltpu.VMEM((_N,), jnp.float32),
        pltpu.VMEM((_N,), jnp.float32),
        pltpu.VMEM((_N,), jnp.float32),
        pltpu.VMEM((_N,), jnp.float32),
        pltpu.VMEM((16,), jnp.float32),
        pltpu.SemaphoreType.DMA,
    ],
)
def _sc_kernel(xb_hbm, sq_hbm, out_hbm, x0_v, x1_v, x2_v, sq_v, frow_v, res_v, sem):
    _sc_body(xb_hbm, sq_hbm, out_hbm, x0_v, x1_v, x2_v, sq_v, frow_v, res_v, sem)


def kernel(pcd):
    xt = jnp.transpose(pcd, (0, 2, 1))
    sq = jnp.sum(xt * xt, axis=-1)
    xb = pcd[:_NB_SC].astype(jnp.bfloat16).astype(jnp.float32)
    sc_parts = _sc_kernel(xb.reshape(-1), sq[:_NB_SC].reshape(-1))
    total = jnp.sum(sc_parts)
    if _NB_SC < _B:
        total = total + _tc_total(xt[_NB_SC:], pcd[_NB_SC:], _B - _NB_SC)
    return total / (_B * _N * 4)


# speedup vs baseline: 2.5653x; 2.5653x over previous
import functools

import jax
import jax.numpy as jnp
from jax import lax
from jax.experimental import pallas as pl
from jax.experimental.pallas import tpu as pltpu
from jax.experimental.pallas import tpu_sc as plsc

_B = 16
_N = 2048
_H = 0.0005
_ROWS = 256

_NB_SC = 4
_NW = 32
_RPW = _NB_SC * _N // _NW


def _tc_body(xr_ref, xc_ref, out_ref):
    b = pl.program_id(0)
    r = pl.program_id(1)

    xr = xr_ref[0]
    xc = xc_ref[0]
    sq_r = jnp.sum(xr * xr, axis=1, keepdims=True)
    sq_c = jnp.sum(xc * xc, axis=0, keepdims=True)
    g = jax.lax.dot_general(
        xr.astype(jnp.bfloat16), xc.astype(jnp.bfloat16), (((1,), (0,)), ((), ())),
        preferred_element_type=jnp.float32,
    )
    d2 = jnp.maximum(sq_r + sq_c - 2.0 * g, 0.0)

    f = jnp.maximum(_H - d2 * d2, 0.0)
    iszero = d2 == 0.0
    n0 = jnp.sum(iszero.astype(jnp.float32), axis=1, keepdims=True)
    fp = jnp.where(iszero, 0.0, f)
    m1 = jnp.max(fp, axis=1, keepdims=True)
    f1 = jnp.where(fp == m1, 0.0, fp)
    m2 = jnp.max(f1, axis=1, keepdims=True)
    f2 = jnp.where(f1 == m2, 0.0, f1)
    m3 = jnp.max(f2, axis=1, keepdims=True)
    f3 = jnp.where(f2 == m3, 0.0, f2)
    m4 = jnp.max(f3, axis=1, keepdims=True)
    f4 = jnp.where(f3 == m4, 0.0, f3)
    m5 = jnp.max(f4, axis=1, keepdims=True)

    nz5 = jnp.minimum(n0, 5.0)
    npos = 5.0 - nz5
    s_pos = (jnp.where(npos >= 1, m1, 0.0) + jnp.where(npos >= 2, m2, 0.0)
             + jnp.where(npos >= 3, m3, 0.0) + jnp.where(npos >= 4, m4, 0.0)
             + jnp.where(npos >= 5, m5, 0.0))
    maxf = jnp.where(n0 > 0, _H, m1)
    contrib = jnp.sum(nz5 * _H + s_pos - maxf)

    @pl.when(jnp.logical_and(b == 0, r == 0))
    def _():
        out_ref[0, 0] = 0.0

    out_ref[0, 0] += contrib


def _tc_total(xt, pcd, nb):
    return pl.pallas_call(
        _tc_body,
        grid=(nb, _N // _ROWS),
        in_specs=[
            pl.BlockSpec((1, _ROWS, 3), lambda b, r: (b, r, 0)),
            pl.BlockSpec((1, 3, _N), lambda b, r: (b, 0, 0)),
        ],
        out_specs=pl.BlockSpec(memory_space=pltpu.SMEM),
        out_shape=jax.ShapeDtypeStruct((1, 1), jnp.float32),
    )(xt, pcd)[0, 0]


def _sc_body(xb_hbm, sq_hbm, out_hbm, x0_v, x1_v, x2_v, sq_v, frow_v, res_v, sem):
    wid = lax.axis_index("s") * 2 + lax.axis_index("c")
    base = wid * _RPW
    b = base // _N
    row0 = base % _N

    xoff = b * 3 * _N
    pltpu.sync_copy(xb_hbm.at[pl.ds(xoff, _N)], x0_v)
    pltpu.sync_copy(xb_hbm.at[pl.ds(xoff + _N, _N)], x1_v)
    pltpu.sync_copy(xb_hbm.at[pl.ds(xoff + 2 * _N, _N)], x2_v)
    pltpu.sync_copy(sq_hbm.at[pl.ds(b * _N, _N)], sq_v)

    def round_chunk(c, acc):
        sl = pl.ds(c * 16, 16)
        for ref in (x0_v, x1_v, x2_v):
            u = plsc.bitcast(ref[sl], jnp.uint32)
            u = (u + jnp.uint32(0x7FFF) + ((u >> jnp.uint32(16)) & jnp.uint32(1))) & jnp.uint32(0xFFFF0000)
            ref[sl] = plsc.bitcast(u, jnp.float32)
        return acc

    lax.fori_loop(0, _N // 16, round_chunk, 0)

    def row_step(i, total):
        n = row0 + i
        n16 = (n // 16) * 16
        onehot = jnp.where(lax.iota(jnp.int32, 16) == n - n16, 1.0, 0.0)
        slq = pl.ds(n16, 16)
        zero16 = jnp.zeros((16,), jnp.float32)

        def bcast(ref):
            return zero16 + lax.reduce_sum(ref[slq] * onehot, (0,))

        xn0 = bcast(x0_v)
        xn1 = bcast(x1_v)
        xn2 = bcast(x2_v)
        sqn = bcast(sq_v)

        def chunk_step(c, carry):
            s_vec, c_vec, z_vec, m_vec = carry
            sl = pl.ds(c * 16, 16)
            g = xn0 * x0_v[sl] + xn1 * x1_v[sl] + xn2 * x2_v[sl]
            d2 = jnp.maximum((sqn + sq_v[sl]) - 2.0 * g, 0.0)
            isz = d2 == 0.0
            f = jnp.maximum(_H - d2 * d2, 0.0)
            fp = jnp.where(isz, 0.0, f)
            frow_v[sl] = fp
            s_vec = s_vec + fp
            c_vec = c_vec + jnp.where(fp > 0.0, 1.0, 0.0)
            z_vec = z_vec + jnp.where(isz, 1.0, 0.0)
            m_vec = jnp.maximum(m_vec, fp)
            return s_vec, c_vec, z_vec, m_vec

        zero = jnp.zeros((16,), jnp.float32)
        s_vec, c_vec, z_vec, m_vec = lax.fori_loop(
            0, _N // 16, chunk_step, (zero, zero, zero, zero))
        s = lax.reduce_sum(s_vec, (0,))
        cnt = lax.reduce_sum(c_vec, (0,))
        n0 = lax.reduce_sum(z_vec, (0,))
        m1 = lax.reduce_max(m_vec, (0,))

        nz5 = jnp.minimum(n0, 5.0)
        npos = 5.0 - nz5

        def fast(_):
            return s

        def slow(_):
            def next_max(prev):
                def pass_step(c, acc):
                    sl = pl.ds(c * 16, 16)
                    v = frow_v[sl]
                    return jnp.maximum(acc, jnp.where(v < prev, v, 0.0))
                mv = lax.fori_loop(0, _N // 16, pass_step, zero)
                return lax.reduce_max(mv, (0,))
            m2 = next_max(m1)
            m3 = next_max(m2)
            m4 = next_max(m3)
            m5 = next_max(m4)
            return (jnp.where(npos >= 1, m1, 0.0) + jnp.where(npos >= 2, m2, 0.0)
                    + jnp.where(npos >= 3, m3, 0.0) + jnp.where(npos >= 4, m4, 0.0)
                    + jnp.where(npos >= 5, m5, 0.0))

        s_pos = lax.cond(cnt > npos, slow, fast, 0)
        maxf = jnp.where(n0 > 0.0, _H, m1)
        return total + (nz5 * _H + s_pos - maxf)

    total = lax.fori_loop(0, _RPW, row_step, jnp.float32(0.0))
    res_v[...] = jnp.where(lax.iota(jnp.int32, 16) == 0, total, 0.0)
    pltpu.sync_copy(res_v, out_hbm.at[pl.ds(wid * 16, 16)])


@functools.partial(
    pl.kernel,
    mesh=plsc.VectorSubcoreMesh(core_axis_name="c", subcore_axis_name="s"),
    out_type=jax.ShapeDtypeStruct((_NW * 16,), jnp.float32),
    compiler_params=pltpu.CompilerParams(needs_layout_passes=False),
    scratch_types=[
        pltpu.VMEM((_N,), jnp.float32),
        pltpu.VMEM((_N,), jnp.float32),
        pltpu.VMEM((_N,), jnp.float32),
        pltpu.VMEM((_N,), jnp.float32),
        pltpu.VMEM((_N,), jnp.float32),
        pltpu.VMEM((16,), jnp.float32),
        pltpu.SemaphoreType.DMA,
    ],
)
def _sc_kernel(xb_hbm, sq_hbm, out_hbm, x0_v, x1_v, x2_v, sq_v, frow_v, res_v, sem):
    _sc_body(xb_hbm, sq_hbm, out_hbm, x0_v, x1_v, x2_v, sq_v, frow_v, res_v, sem)


def kernel(pcd):
    xt = jnp.transpose(pcd, (0, 2, 1))
    sq = jnp.sum(xt * xt, axis=-1)
    xb = pcd[:_NB_SC].astype(jnp.bfloat16).astype(jnp.float32)
    sc_parts = _sc_kernel(xb.reshape(-1), sq[:_NB_SC].reshape(-1))
    total = jnp.sum(sc_parts)
    if _NB_SC < _B:
        total = total + _tc_total(xt[_NB_SC:], pcd[_NB_SC:], _B - _NB_SC)
    return total / (_B * _N * 4)


# speedup vs baseline: 2.8142x; 1.0970x over previous
import functools

import jax
import jax.numpy as jnp
from jax import lax
from jax.experimental import pallas as pl
from jax.experimental.pallas import tpu as pltpu
from jax.experimental.pallas import tpu_sc as plsc

_B = 16
_N = 2048
_H = 0.0005
_ROWS = 256

_NB_SC = 4
_NW = 32
_RPW = _NB_SC * _N // _NW


def _tc_body(xr_ref, xc_ref, out_ref):
    b = pl.program_id(0)
    r = pl.program_id(1)

    xr = xr_ref[0]
    xc = xc_ref[0]
    sq_r = jnp.sum(xr * xr, axis=1, keepdims=True)
    sq_c = jnp.sum(xc * xc, axis=0, keepdims=True)
    g = jax.lax.dot_general(
        xr.astype(jnp.bfloat16), xc.astype(jnp.bfloat16), (((1,), (0,)), ((), ())),
        preferred_element_type=jnp.float32,
    )
    d2 = jnp.maximum(sq_r + sq_c - 2.0 * g, 0.0)

    f = jnp.maximum(_H - d2 * d2, 0.0)
    iszero = d2 == 0.0
    n0 = jnp.sum(iszero.astype(jnp.float32), axis=1, keepdims=True)
    fp = jnp.where(iszero, 0.0, f)
    m1 = jnp.max(fp, axis=1, keepdims=True)
    f1 = jnp.where(fp == m1, 0.0, fp)
    m2 = jnp.max(f1, axis=1, keepdims=True)
    f2 = jnp.where(f1 == m2, 0.0, f1)
    m3 = jnp.max(f2, axis=1, keepdims=True)
    f3 = jnp.where(f2 == m3, 0.0, f2)
    m4 = jnp.max(f3, axis=1, keepdims=True)
    f4 = jnp.where(f3 == m4, 0.0, f3)
    m5 = jnp.max(f4, axis=1, keepdims=True)

    nz5 = jnp.minimum(n0, 5.0)
    npos = 5.0 - nz5
    s_pos = (jnp.where(npos >= 1, m1, 0.0) + jnp.where(npos >= 2, m2, 0.0)
             + jnp.where(npos >= 3, m3, 0.0) + jnp.where(npos >= 4, m4, 0.0)
             + jnp.where(npos >= 5, m5, 0.0))
    maxf = jnp.where(n0 > 0, _H, m1)
    contrib = jnp.sum(nz5 * _H + s_pos - maxf)

    @pl.when(jnp.logical_and(b == 0, r == 0))
    def _():
        out_ref[0, 0] = 0.0

    out_ref[0, 0] += contrib


def _tc_total(xt, pcd, nb):
    return pl.pallas_call(
        _tc_body,
        grid=(nb, _N // _ROWS),
        in_specs=[
            pl.BlockSpec((1, _ROWS, 3), lambda b, r: (b, r, 0)),
            pl.BlockSpec((1, 3, _N), lambda b, r: (b, 0, 0)),
        ],
        out_specs=pl.BlockSpec(memory_space=pltpu.SMEM),
        out_shape=jax.ShapeDtypeStruct((1, 1), jnp.float32),
    )(xt, pcd)[0, 0]


def _sc_body(xb_hbm, out_hbm, x0_v, x1_v, x2_v, sq_v, frow_v, res_v, sem):
    wid = lax.axis_index("s") * 2 + lax.axis_index("c")
    base = wid * _RPW
    b = base // _N
    row0 = base % _N

    xoff = b * 3 * _N
    pltpu.sync_copy(xb_hbm.at[pl.ds(xoff, _N)], x0_v)
    pltpu.sync_copy(xb_hbm.at[pl.ds(xoff + _N, _N)], x1_v)
    pltpu.sync_copy(xb_hbm.at[pl.ds(xoff + 2 * _N, _N)], x2_v)
    def prep_chunk(c, acc):
        sl = pl.ds(c * 16, 16)
        a0, a1, a2 = x0_v[sl], x1_v[sl], x2_v[sl]
        sq_v[sl] = a0 * a0 + a1 * a1 + a2 * a2
        for ref, a in ((x0_v, a0), (x1_v, a1), (x2_v, a2)):
            u = plsc.bitcast(a, jnp.uint32)
            u = (u + jnp.uint32(0x7FFF) + ((u >> jnp.uint32(16)) & jnp.uint32(1))) & jnp.uint32(0xFFFF0000)
            ref[sl] = plsc.bitcast(u, jnp.float32)
        return acc

    lax.fori_loop(0, _N // 16, prep_chunk, 0)

    def row_step(i, total):
        n = row0 + i
        n16 = (n // 16) * 16
        onehot = jnp.where(lax.iota(jnp.int32, 16) == n - n16, 1.0, 0.0)
        slq = pl.ds(n16, 16)
        zero16 = jnp.zeros((16,), jnp.float32)

        def bcast(ref):
            return zero16 + lax.reduce_sum(ref[slq] * onehot, (0,))

        xn0 = bcast(x0_v)
        xn1 = bcast(x1_v)
        xn2 = bcast(x2_v)
        sqn = bcast(sq_v)

        def chunk_step(c, carry):
            s_vec, c_vec, z_vec, m_vec = carry
            sl = pl.ds(c * 16, 16)
            g = xn0 * x0_v[sl] + xn1 * x1_v[sl] + xn2 * x2_v[sl]
            d2 = jnp.maximum((sqn + sq_v[sl]) - 2.0 * g, 0.0)
            isz = d2 == 0.0
            f = jnp.maximum(_H - d2 * d2, 0.0)
            fp = jnp.where(isz, 0.0, f)
            frow_v[sl] = fp
            s_vec = s_vec + fp
            c_vec = c_vec + jnp.where(fp > 0.0, 1.0, 0.0)
            z_vec = z_vec + jnp.where(isz, 1.0, 0.0)
            m_vec = jnp.maximum(m_vec, fp)
            return s_vec, c_vec, z_vec, m_vec

        zero = jnp.zeros((16,), jnp.float32)
        s_vec, c_vec, z_vec, m_vec = lax.fori_loop(
            0, _N // 16, chunk_step, (zero, zero, zero, zero))
        s = lax.reduce_sum(s_vec, (0,))
        cnt = lax.reduce_sum(c_vec, (0,))
        n0 = lax.reduce_sum(z_vec, (0,))
        m1 = lax.reduce_max(m_vec, (0,))

        nz5 = jnp.minimum(n0, 5.0)
        npos = 5.0 - nz5

        def fast(_):
            return s

        def slow(_):
            def next_max(prev):
                def pass_step(c, acc):
                    sl = pl.ds(c * 16, 16)
                    v = frow_v[sl]
                    return jnp.maximum(acc, jnp.where(v < prev, v, 0.0))
                mv = lax.fori_loop(0, _N // 16, pass_step, zero)
                return lax.reduce_max(mv, (0,))
            m2 = next_max(m1)
            m3 = next_max(m2)
            m4 = next_max(m3)
            m5 = next_max(m4)
            return (jnp.where(npos >= 1, m1, 0.0) + jnp.where(npos >= 2, m2, 0.0)
                    + jnp.where(npos >= 3, m3, 0.0) + jnp.where(npos >= 4, m4, 0.0)
                    + jnp.where(npos >= 5, m5, 0.0))

        s_pos = lax.cond(cnt > npos, slow, fast, 0)
        maxf = jnp.where(n0 > 0.0, _H, m1)
        return total + (nz5 * _H + s_pos - maxf)

    total = lax.fori_loop(0, _RPW, row_step, jnp.float32(0.0))
    res_v[...] = jnp.where(lax.iota(jnp.int32, 16) == 0, total, 0.0)
    pltpu.sync_copy(res_v, out_hbm.at[pl.ds(wid * 16, 16)])


@functools.partial(
    pl.kernel,
    mesh=plsc.VectorSubcoreMesh(core_axis_name="c", subcore_axis_name="s"),
    out_type=jax.ShapeDtypeStruct((_NW * 16,), jnp.float32),
    compiler_params=pltpu.CompilerParams(needs_layout_passes=False),
    scratch_types=[
        pltpu.VMEM((_N,), jnp.float32),
        pltpu.VMEM((_N,), jnp.float32),
        pltpu.VMEM((_N,), jnp.float32),
        pltpu.VMEM((_N,), jnp.float32),
        pltpu.VMEM((_N,), jnp.float32),
        pltpu.VMEM((16,), jnp.float32),
        pltpu.SemaphoreType.DMA,
    ],
)
def _sc_kernel(xb_hbm, out_hbm, x0_v, x1_v, x2_v, sq_v, frow_v, res_v, sem):
    _sc_body(xb_hbm, out_hbm, x0_v, x1_v, x2_v, sq_v, frow_v, res_v, sem)


def kernel(pcd):
    sc_parts = _sc_kernel(pcd[:_NB_SC].reshape(-1))
    total = jnp.sum(sc_parts)
    if _NB_SC < _B:
        xt = jnp.transpose(pcd[_NB_SC:], (0, 2, 1))
        total = total + _tc_total(xt, pcd[_NB_SC:], _B - _NB_SC)
    return total / (_B * _N * 4)


# speedup vs baseline: 3.0022x; 1.0668x over previous
import functools

import jax
import jax.numpy as jnp
from jax import lax
from jax.experimental import pallas as pl
from jax.experimental.pallas import tpu as pltpu
from jax.experimental.pallas import tpu_sc as plsc

_B = 16
_N = 2048
_H = 0.0005
_ROWS = 256

_NB_SC = 4
_NW = 32
_RPW = _NB_SC * _N // _NW


def _tc_body(xr_ref, xc_ref, out_ref):
    b = pl.program_id(0)
    r = pl.program_id(1)

    xr = xr_ref[0]
    xc = xc_ref[0]
    sq_r = jnp.sum(xr * xr, axis=1, keepdims=True)
    sq_c = jnp.sum(xc * xc, axis=0, keepdims=True)
    g = jax.lax.dot_general(
        xr.astype(jnp.bfloat16), xc.astype(jnp.bfloat16), (((1,), (0,)), ((), ())),
        preferred_element_type=jnp.float32,
    )
    d2 = jnp.maximum(sq_r + sq_c - 2.0 * g, 0.0)

    inf = jnp.float32(jnp.inf)
    iszero = d2 == 0.0
    n0 = jnp.sum(iszero.astype(jnp.float32), axis=1, keepdims=True)
    dp = jnp.where(iszero, inf, d2)
    m1 = jnp.min(dp, axis=1, keepdims=True)
    e1 = jnp.where(dp == m1, inf, dp)
    m2 = jnp.min(e1, axis=1, keepdims=True)
    e2 = jnp.where(e1 == m2, inf, e1)
    m3 = jnp.min(e2, axis=1, keepdims=True)
    e3 = jnp.where(e2 == m3, inf, e2)
    m4 = jnp.min(e3, axis=1, keepdims=True)
    m5 = jnp.min(jnp.where(e3 == m4, inf, e3), axis=1, keepdims=True)

    f1 = jnp.maximum(_H - m1 * m1, 0.0)
    f2 = jnp.maximum(_H - m2 * m2, 0.0)
    f3 = jnp.maximum(_H - m3 * m3, 0.0)
    f4 = jnp.maximum(_H - m4 * m4, 0.0)
    f5 = jnp.maximum(_H - m5 * m5, 0.0)

    nz5 = jnp.minimum(n0, 5.0)
    npos = 5.0 - nz5
    s_pos = (jnp.where(npos >= 1, f1, 0.0) + jnp.where(npos >= 2, f2, 0.0)
             + jnp.where(npos >= 3, f3, 0.0) + jnp.where(npos >= 4, f4, 0.0)
             + jnp.where(npos >= 5, f5, 0.0))
    maxf = jnp.where(n0 > 0, _H, f1)
    contrib = jnp.sum(nz5 * _H + s_pos - maxf)

    @pl.when(jnp.logical_and(b == 0, r == 0))
    def _():
        out_ref[0, 0] = 0.0

    out_ref[0, 0] += contrib


def _tc_total(xt, pcd, nb):
    return pl.pallas_call(
        _tc_body,
        grid=(nb, _N // _ROWS),
        in_specs=[
            pl.BlockSpec((1, _ROWS, 3), lambda b, r: (b, r, 0)),
            pl.BlockSpec((1, 3, _N), lambda b, r: (b, 0, 0)),
        ],
        out_specs=pl.BlockSpec(memory_space=pltpu.SMEM),
        out_shape=jax.ShapeDtypeStruct((1, 1), jnp.float32),
    )(xt, pcd)[0, 0]


def _sc_body(xb_hbm, out_hbm, x0_v, x1_v, x2_v, sq_v, frow_v, res_v, sem):
    wid = lax.axis_index("s") * 2 + lax.axis_index("c")
    base = wid * _RPW
    b = base // _N
    row0 = base % _N

    xoff = b * 3 * _N
    pltpu.sync_copy(xb_hbm.at[pl.ds(xoff, _N)], x0_v)
    pltpu.sync_copy(xb_hbm.at[pl.ds(xoff + _N, _N)], x1_v)
    pltpu.sync_copy(xb_hbm.at[pl.ds(xoff + 2 * _N, _N)], x2_v)
    def prep_chunk(c, acc):
        sl = pl.ds(c * 16, 16)
        a0, a1, a2 = x0_v[sl], x1_v[sl], x2_v[sl]
        sq_v[sl] = a0 * a0 + a1 * a1 + a2 * a2
        for ref, a in ((x0_v, a0), (x1_v, a1), (x2_v, a2)):
            u = plsc.bitcast(a, jnp.uint32)
            u = (u + jnp.uint32(0x7FFF) + ((u >> jnp.uint32(16)) & jnp.uint32(1))) & jnp.uint32(0xFFFF0000)
            ref[sl] = plsc.bitcast(u, jnp.float32)
        return acc

    lax.fori_loop(0, _N // 16, prep_chunk, 0)

    def row_step(i, total):
        n = row0 + i
        n16 = (n // 16) * 16
        onehot = jnp.where(lax.iota(jnp.int32, 16) == n - n16, 1.0, 0.0)
        slq = pl.ds(n16, 16)
        zero16 = jnp.zeros((16,), jnp.float32)

        def bcast(ref):
            return zero16 + lax.reduce_sum(ref[slq] * onehot, (0,))

        xn0 = bcast(x0_v)
        xn1 = bcast(x1_v)
        xn2 = bcast(x2_v)
        sqn = bcast(sq_v)

        def chunk_step(c, carry):
            s_vec, c_vec, z_vec, m_vec = carry
            sl = pl.ds(c * 16, 16)
            g = xn0 * x0_v[sl] + xn1 * x1_v[sl] + xn2 * x2_v[sl]
            d2 = jnp.maximum((sqn + sq_v[sl]) - 2.0 * g, 0.0)
            isz = d2 == 0.0
            f = jnp.maximum(_H - d2 * d2, 0.0)
            fp = jnp.where(isz, 0.0, f)
            frow_v[sl] = fp
            s_vec = s_vec + fp
            c_vec = c_vec + jnp.where(fp > 0.0, 1.0, 0.0)
            z_vec = z_vec + jnp.where(isz, 1.0, 0.0)
            m_vec = jnp.maximum(m_vec, fp)
            return s_vec, c_vec, z_vec, m_vec

        zero = jnp.zeros((16,), jnp.float32)
        s_vec, c_vec, z_vec, m_vec = lax.fori_loop(
            0, _N // 16, chunk_step, (zero, zero, zero, zero))
        s = lax.reduce_sum(s_vec, (0,))
        cnt = lax.reduce_sum(c_vec, (0,))
        n0 = lax.reduce_sum(z_vec, (0,))
        m1 = lax.reduce_max(m_vec, (0,))

        nz5 = jnp.minimum(n0, 5.0)
        npos = 5.0 - nz5

        def fast(_):
            return s

        def slow(_):
            def next_max(prev):
                def pass_step(c, acc):
                    sl = pl.ds(c * 16, 16)
                    v = frow_v[sl]
                    return jnp.maximum(acc, jnp.where(v < prev, v, 0.0))
                mv = lax.fori_loop(0, _N // 16, pass_step, zero)
                return lax.reduce_max(mv, (0,))
            m2 = next_max(m1)
            m3 = next_max(m2)
            m4 = next_max(m3)
            m5 = next_max(m4)
            return (jnp.where(npos >= 1, m1, 0.0) + jnp.where(npos >= 2, m2, 0.0)
                    + jnp.where(npos >= 3, m3, 0.0) + jnp.where(npos >= 4, m4, 0.0)
                    + jnp.where(npos >= 5, m5, 0.0))

        s_pos = lax.cond(cnt > npos, slow, fast, 0)
        maxf = jnp.where(n0 > 0.0, _H, m1)
        return total + (nz5 * _H + s_pos - maxf)

    total = lax.fori_loop(0, _RPW, row_step, jnp.float32(0.0))
    res_v[...] = jnp.where(lax.iota(jnp.int32, 16) == 0, total, 0.0)
    pltpu.sync_copy(res_v, out_hbm.at[pl.ds(wid * 16, 16)])


@functools.partial(
    pl.kernel,
    mesh=plsc.VectorSubcoreMesh(core_axis_name="c", subcore_axis_name="s"),
    out_type=jax.ShapeDtypeStruct((_NW * 16,), jnp.float32),
    compiler_params=pltpu.CompilerParams(needs_layout_passes=False),
    scratch_types=[
        pltpu.VMEM((_N,), jnp.float32),
        pltpu.VMEM((_N,), jnp.float32),
        pltpu.VMEM((_N,), jnp.float32),
        pltpu.VMEM((_N,), jnp.float32),
        pltpu.VMEM((_N,), jnp.float32),
        pltpu.VMEM((16,), jnp.float32),
        pltpu.SemaphoreType.DMA,
    ],
)
def _sc_kernel(xb_hbm, out_hbm, x0_v, x1_v, x2_v, sq_v, frow_v, res_v, sem):
    _sc_body(xb_hbm, out_hbm, x0_v, x1_v, x2_v, sq_v, frow_v, res_v, sem)


def kernel(pcd):
    sc_parts = _sc_kernel(pcd[:_NB_SC].reshape(-1))
    total = jnp.sum(sc_parts)
    if _NB_SC < _B:
        xt = jnp.transpose(pcd[_NB_SC:], (0, 2, 1))
        total = total + _tc_total(xt, pcd[_NB_SC:], _B - _NB_SC)
    return total / (_B * _N * 4)


# speedup vs baseline: 3.1439x; 1.0472x over previous
import functools

import jax
import jax.numpy as jnp
from jax import lax
from jax.experimental import pallas as pl
from jax.experimental.pallas import tpu as pltpu
from jax.experimental.pallas import tpu_sc as plsc

_B = 16
_N = 2048
_H = 0.0005
_ROWS = 512

_NB_SC = 4
_NW = 32
_RPW = _NB_SC * _N // _NW


def _tc_body(xr_ref, xc_ref, out_ref):
    b = pl.program_id(0)
    r = pl.program_id(1)

    xr = xr_ref[0]
    xc = xc_ref[0]
    sq_r = jnp.sum(xr * xr, axis=1, keepdims=True)
    sq_c = jnp.sum(xc * xc, axis=0, keepdims=True)
    g = jax.lax.dot_general(
        xr.astype(jnp.bfloat16), xc.astype(jnp.bfloat16), (((1,), (0,)), ((), ())),
        preferred_element_type=jnp.float32,
    )
    d2 = jnp.maximum(sq_r + sq_c - 2.0 * g, 0.0)

    inf = jnp.float32(jnp.inf)
    iszero = d2 == 0.0
    n0 = jnp.sum(iszero.astype(jnp.float32), axis=1, keepdims=True)
    dp = jnp.where(iszero, inf, d2)
    m1 = jnp.min(dp, axis=1, keepdims=True)
    e1 = jnp.where(dp == m1, inf, dp)
    m2 = jnp.min(e1, axis=1, keepdims=True)
    e2 = jnp.where(e1 == m2, inf, e1)
    m3 = jnp.min(e2, axis=1, keepdims=True)
    e3 = jnp.where(e2 == m3, inf, e2)
    m4 = jnp.min(e3, axis=1, keepdims=True)
    m5 = jnp.min(jnp.where(e3 == m4, inf, e3), axis=1, keepdims=True)

    f1 = jnp.maximum(_H - m1 * m1, 0.0)
    f2 = jnp.maximum(_H - m2 * m2, 0.0)
    f3 = jnp.maximum(_H - m3 * m3, 0.0)
    f4 = jnp.maximum(_H - m4 * m4, 0.0)
    f5 = jnp.maximum(_H - m5 * m5, 0.0)

    nz5 = jnp.minimum(n0, 5.0)
    npos = 5.0 - nz5
    s_pos = (jnp.where(npos >= 1, f1, 0.0) + jnp.where(npos >= 2, f2, 0.0)
             + jnp.where(npos >= 3, f3, 0.0) + jnp.where(npos >= 4, f4, 0.0)
             + jnp.where(npos >= 5, f5, 0.0))
    maxf = jnp.where(n0 > 0, _H, f1)
    contrib = jnp.sum(nz5 * _H + s_pos - maxf)

    @pl.when(jnp.logical_and(b == 0, r == 0))
    def _():
        out_ref[0, 0] = 0.0

    out_ref[0, 0] += contrib


def _tc_total(xt, pcd, nb):
    return pl.pallas_call(
        _tc_body,
        grid=(nb, _N // _ROWS),
        in_specs=[
            pl.BlockSpec((1, _ROWS, 3), lambda b, r: (b, r, 0)),
            pl.BlockSpec((1, 3, _N), lambda b, r: (b, 0, 0)),
        ],
        out_specs=pl.BlockSpec(memory_space=pltpu.SMEM),
        out_shape=jax.ShapeDtypeStruct((1, 1), jnp.float32),
    )(xt, pcd)[0, 0]


def _sc_body(xb_hbm, out_hbm, x0_v, x1_v, x2_v, sq_v, frow_v, res_v, sem):
    wid = lax.axis_index("s") * 2 + lax.axis_index("c")
    base = wid * _RPW
    b = base // _N
    row0 = base % _N

    xoff = b * 3 * _N
    pltpu.sync_copy(xb_hbm.at[pl.ds(xoff, _N)], x0_v)
    pltpu.sync_copy(xb_hbm.at[pl.ds(xoff + _N, _N)], x1_v)
    pltpu.sync_copy(xb_hbm.at[pl.ds(xoff + 2 * _N, _N)], x2_v)
    def prep_chunk(c, acc):
        sl = pl.ds(c * 16, 16)
        a0, a1, a2 = x0_v[sl], x1_v[sl], x2_v[sl]
        sq_v[sl] = a0 * a0 + a1 * a1 + a2 * a2
        for ref, a in ((x0_v, a0), (x1_v, a1), (x2_v, a2)):
            u = plsc.bitcast(a, jnp.uint32)
            u = (u + jnp.uint32(0x7FFF) + ((u >> jnp.uint32(16)) & jnp.uint32(1))) & jnp.uint32(0xFFFF0000)
            ref[sl] = plsc.bitcast(u, jnp.float32)
        return acc

    lax.fori_loop(0, _N // 16, prep_chunk, 0)

    def row_step(i, total):
        n = row0 + i
        n16 = (n // 16) * 16
        onehot = jnp.where(lax.iota(jnp.int32, 16) == n - n16, 1.0, 0.0)
        slq = pl.ds(n16, 16)
        zero16 = jnp.zeros((16,), jnp.float32)

        def bcast(ref):
            return zero16 + lax.reduce_sum(ref[slq] * onehot, (0,))

        xn0 = bcast(x0_v)
        xn1 = bcast(x1_v)
        xn2 = bcast(x2_v)
        sqn = bcast(sq_v)

        def chunk_step(c, carry):
            s_vec, c_vec, z_vec, m_vec = carry
            sl = pl.ds(c * 16, 16)
            g = xn0 * x0_v[sl] + xn1 * x1_v[sl] + xn2 * x2_v[sl]
            d2 = jnp.maximum((sqn + sq_v[sl]) - 2.0 * g, 0.0)
            isz = d2 == 0.0
            f = jnp.maximum(_H - d2 * d2, 0.0)
            fp = jnp.where(isz, 0.0, f)
            frow_v[sl] = fp
            s_vec = s_vec + fp
            c_vec = c_vec + jnp.where(fp > 0.0, 1.0, 0.0)
            z_vec = z_vec + jnp.where(isz, 1.0, 0.0)
            m_vec = jnp.maximum(m_vec, fp)
            return s_vec, c_vec, z_vec, m_vec

        zero = jnp.zeros((16,), jnp.float32)
        s_vec, c_vec, z_vec, m_vec = lax.fori_loop(
            0, _N // 16, chunk_step, (zero, zero, zero, zero))
        s = lax.reduce_sum(s_vec, (0,))
        cnt = lax.reduce_sum(c_vec, (0,))
        n0 = lax.reduce_sum(z_vec, (0,))
        m1 = lax.reduce_max(m_vec, (0,))

        nz5 = jnp.minimum(n0, 5.0)
        npos = 5.0 - nz5

        def fast(_):
            return s

        def slow(_):
            def next_max(prev):
                def pass_step(c, acc):
                    sl = pl.ds(c * 16, 16)
                    v = frow_v[sl]
                    return jnp.maximum(acc, jnp.where(v < prev, v, 0.0))
                mv = lax.fori_loop(0, _N // 16, pass_step, zero)
                return lax.reduce_max(mv, (0,))
            m2 = next_max(m1)
            m3 = next_max(m2)
            m4 = next_max(m3)
            m5 = next_max(m4)
            return (jnp.where(npos >= 1, m1, 0.0) + jnp.where(npos >= 2, m2, 0.0)
                    + jnp.where(npos >= 3, m3, 0.0) + jnp.where(npos >= 4, m4, 0.0)
                    + jnp.where(npos >= 5, m5, 0.0))

        s_pos = lax.cond(cnt > npos, slow, fast, 0)
        maxf = jnp.where(n0 > 0.0, _H, m1)
        return total + (nz5 * _H + s_pos - maxf)

    total = lax.fori_loop(0, _RPW, row_step, jnp.float32(0.0))
    res_v[...] = jnp.where(lax.iota(jnp.int32, 16) == 0, total, 0.0)
    pltpu.sync_copy(res_v, out_hbm.at[pl.ds(wid * 16, 16)])


@functools.partial(
    pl.kernel,
    mesh=plsc.VectorSubcoreMesh(core_axis_name="c", subcore_axis_name="s"),
    out_type=jax.ShapeDtypeStruct((_NW * 16,), jnp.float32),
    compiler_params=pltpu.CompilerParams(needs_layout_passes=False),
    scratch_types=[
        pltpu.VMEM((_N,), jnp.float32),
        pltpu.VMEM((_N,), jnp.float32),
        pltpu.VMEM((_N,), jnp.float32),
        pltpu.VMEM((_N,), jnp.float32),
        pltpu.VMEM((_N,), jnp.float32),
        pltpu.VMEM((16,), jnp.float32),
        pltpu.SemaphoreType.DMA,
    ],
)
def _sc_kernel(xb_hbm, out_hbm, x0_v, x1_v, x2_v, sq_v, frow_v, res_v, sem):
    _sc_body(xb_hbm, out_hbm, x0_v, x1_v, x2_v, sq_v, frow_v, res_v, sem)


def kernel(pcd):
    sc_parts = _sc_kernel(pcd[:_NB_SC].reshape(-1))
    total = jnp.sum(sc_parts)
    if _NB_SC < _B:
        xt = jnp.transpose(pcd[_NB_SC:], (0, 2, 1))
        total = total + _tc_total(xt, pcd[_NB_SC:], _B - _NB_SC)
    return total / (_B * _N * 4)


# speedup vs baseline: 3.2514x; 1.0342x over previous
import functools

import jax
import jax.numpy as jnp
from jax import lax
from jax.experimental import pallas as pl
from jax.experimental.pallas import tpu as pltpu
from jax.experimental.pallas import tpu_sc as plsc

_B = 16
_N = 2048
_H = 0.0005
_ROWS = 1024

_NB_SC = 4
_NW = 32
_RPW = _NB_SC * _N // _NW


def _tc_body(xr_ref, xc_ref, out_ref):
    b = pl.program_id(0)
    r = pl.program_id(1)

    xr = xr_ref[0]
    xc = xc_ref[0]
    sq_r = jnp.sum(xr * xr, axis=1, keepdims=True)
    sq_c = jnp.sum(xc * xc, axis=0, keepdims=True)
    g = jax.lax.dot_general(
        xr.astype(jnp.bfloat16), xc.astype(jnp.bfloat16), (((1,), (0,)), ((), ())),
        preferred_element_type=jnp.float32,
    )
    d2 = jnp.maximum(sq_r + sq_c - 2.0 * g, 0.0)

    inf = jnp.float32(jnp.inf)
    iszero = d2 == 0.0
    n0 = jnp.sum(iszero.astype(jnp.float32), axis=1, keepdims=True)
    dp = jnp.where(iszero, inf, d2)
    m1 = jnp.min(dp, axis=1, keepdims=True)
    e1 = jnp.where(dp == m1, inf, dp)
    m2 = jnp.min(e1, axis=1, keepdims=True)
    e2 = jnp.where(e1 == m2, inf, e1)
    m3 = jnp.min(e2, axis=1, keepdims=True)
    e3 = jnp.where(e2 == m3, inf, e2)
    m4 = jnp.min(e3, axis=1, keepdims=True)
    m5 = jnp.min(jnp.where(e3 == m4, inf, e3), axis=1, keepdims=True)

    f1 = jnp.maximum(_H - m1 * m1, 0.0)
    f2 = jnp.maximum(_H - m2 * m2, 0.0)
    f3 = jnp.maximum(_H - m3 * m3, 0.0)
    f4 = jnp.maximum(_H - m4 * m4, 0.0)
    f5 = jnp.maximum(_H - m5 * m5, 0.0)

    nz5 = jnp.minimum(n0, 5.0)
    npos = 5.0 - nz5
    s_pos = (jnp.where(npos >= 1, f1, 0.0) + jnp.where(npos >= 2, f2, 0.0)
             + jnp.where(npos >= 3, f3, 0.0) + jnp.where(npos >= 4, f4, 0.0)
             + jnp.where(npos >= 5, f5, 0.0))
    maxf = jnp.where(n0 > 0, _H, f1)
    contrib = jnp.sum(nz5 * _H + s_pos - maxf)

    @pl.when(jnp.logical_and(b == 0, r == 0))
    def _():
        out_ref[0, 0] = 0.0

    out_ref[0, 0] += contrib


def _tc_total(xt, pcd, nb):
    return pl.pallas_call(
        _tc_body,
        grid=(nb, _N // _ROWS),
        in_specs=[
            pl.BlockSpec((1, _ROWS, 3), lambda b, r: (b, r, 0)),
            pl.BlockSpec((1, 3, _N), lambda b, r: (b, 0, 0)),
        ],
        out_specs=pl.BlockSpec(memory_space=pltpu.SMEM),
        out_shape=jax.ShapeDtypeStruct((1, 1), jnp.float32),
    )(xt, pcd)[0, 0]


def _sc_body(xb_hbm, out_hbm, x0_v, x1_v, x2_v, sq_v, frow_v, res_v, sem):
    wid = lax.axis_index("s") * 2 + lax.axis_index("c")
    base = wid * _RPW
    b = base // _N
    row0 = base % _N

    xoff = b * 3 * _N
    pltpu.sync_copy(xb_hbm.at[pl.ds(xoff, _N)], x0_v)
    pltpu.sync_copy(xb_hbm.at[pl.ds(xoff + _N, _N)], x1_v)
    pltpu.sync_copy(xb_hbm.at[pl.ds(xoff + 2 * _N, _N)], x2_v)
    def prep_chunk(c, acc):
        sl = pl.ds(c * 16, 16)
        a0, a1, a2 = x0_v[sl], x1_v[sl], x2_v[sl]
        sq_v[sl] = a0 * a0 + a1 * a1 + a2 * a2
        for ref, a in ((x0_v, a0), (x1_v, a1), (x2_v, a2)):
            u = plsc.bitcast(a, jnp.uint32)
            u = (u + jnp.uint32(0x7FFF) + ((u >> jnp.uint32(16)) & jnp.uint32(1))) & jnp.uint32(0xFFFF0000)
            ref[sl] = plsc.bitcast(u, jnp.float32)
        return acc

    lax.fori_loop(0, _N // 16, prep_chunk, 0)

    def row_step(i, total):
        n = row0 + i
        n16 = (n // 16) * 16
        onehot = jnp.where(lax.iota(jnp.int32, 16) == n - n16, 1.0, 0.0)
        slq = pl.ds(n16, 16)
        zero16 = jnp.zeros((16,), jnp.float32)

        def bcast(ref):
            return zero16 + lax.reduce_sum(ref[slq] * onehot, (0,))

        xn0 = bcast(x0_v)
        xn1 = bcast(x1_v)
        xn2 = bcast(x2_v)
        sqn = bcast(sq_v)

        def chunk_step(c, carry):
            s_vec, c_vec, z_vec, m_vec = carry
            sl = pl.ds(c * 16, 16)
            g = xn0 * x0_v[sl] + xn1 * x1_v[sl] + xn2 * x2_v[sl]
            d2 = jnp.maximum((sqn + sq_v[sl]) - 2.0 * g, 0.0)
            isz = d2 == 0.0
            f = jnp.maximum(_H - d2 * d2, 0.0)
            fp = jnp.where(isz, 0.0, f)
            frow_v[sl] = fp
            s_vec = s_vec + fp
            c_vec = c_vec + jnp.where(fp > 0.0, 1.0, 0.0)
            z_vec = z_vec + jnp.where(isz, 1.0, 0.0)
            m_vec = jnp.maximum(m_vec, fp)
            return s_vec, c_vec, z_vec, m_vec

        zero = jnp.zeros((16,), jnp.float32)
        s_vec, c_vec, z_vec, m_vec = lax.fori_loop(
            0, _N // 16, chunk_step, (zero, zero, zero, zero))
        s = lax.reduce_sum(s_vec, (0,))
        cnt = lax.reduce_sum(c_vec, (0,))
        n0 = lax.reduce_sum(z_vec, (0,))
        m1 = lax.reduce_max(m_vec, (0,))

        nz5 = jnp.minimum(n0, 5.0)
        npos = 5.0 - nz5

        def fast(_):
            return s

        def slow(_):
            def next_max(prev):
                def pass_step(c, acc):
                    sl = pl.ds(c * 16, 16)
                    v = frow_v[sl]
                    return jnp.maximum(acc, jnp.where(v < prev, v, 0.0))
                mv = lax.fori_loop(0, _N // 16, pass_step, zero)
                return lax.reduce_max(mv, (0,))
            m2 = next_max(m1)
            m3 = next_max(m2)
            m4 = next_max(m3)
            m5 = next_max(m4)
            return (jnp.where(npos >= 1, m1, 0.0) + jnp.where(npos >= 2, m2, 0.0)
                    + jnp.where(npos >= 3, m3, 0.0) + jnp.where(npos >= 4, m4, 0.0)
                    + jnp.where(npos >= 5, m5, 0.0))

        s_pos = lax.cond(cnt > npos, slow, fast, 0)
        maxf = jnp.where(n0 > 0.0, _H, m1)
        return total + (nz5 * _H + s_pos - maxf)

    total = lax.fori_loop(0, _RPW, row_step, jnp.float32(0.0))
    res_v[...] = jnp.where(lax.iota(jnp.int32, 16) == 0, total, 0.0)
    pltpu.sync_copy(res_v, out_hbm.at[pl.ds(wid * 16, 16)])


@functools.partial(
    pl.kernel,
    mesh=plsc.VectorSubcoreMesh(core_axis_name="c", subcore_axis_name="s"),
    out_type=jax.ShapeDtypeStruct((_NW * 16,), jnp.float32),
    compiler_params=pltpu.CompilerParams(needs_layout_passes=False),
    scratch_types=[
        pltpu.VMEM((_N,), jnp.float32),
        pltpu.VMEM((_N,), jnp.float32),
        pltpu.VMEM((_N,), jnp.float32),
        pltpu.VMEM((_N,), jnp.float32),
        pltpu.VMEM((_N,), jnp.float32),
        pltpu.VMEM((16,), jnp.float32),
        pltpu.SemaphoreType.DMA,
    ],
)
def _sc_kernel(xb_hbm, out_hbm, x0_v, x1_v, x2_v, sq_v, frow_v, res_v, sem):
    _sc_body(xb_hbm, out_hbm, x0_v, x1_v, x2_v, sq_v, frow_v, res_v, sem)


def kernel(pcd):
    sc_parts = _sc_kernel(pcd[:_NB_SC].reshape(-1))
    total = jnp.sum(sc_parts)
    if _NB_SC < _B:
        xt = jnp.transpose(pcd[_NB_SC:], (0, 2, 1))
        total = total + _tc_total(xt, pcd[_NB_SC:], _B - _NB_SC)
    return total / (_B * _N * 4)
